# baseline (speedup 1.0000x reference)
import functools

import jax
import jax.numpy as jnp
from jax import lax
from jax.experimental import pallas as pl
from jax.experimental.pallas import tpu as pltpu
from jax.experimental.pallas import tpu_sc as plsc

N = 286
NPAD = 288
NCH_N = NPAD // 16
E = 9152
NCH_E = E // 16
NSUB = 16
BIG = 36
NBIG = NCH_E - NSUB * (BIG - 1)
EPT = BIG * 16

ACC_COLS = 48
COL_B = 24


def _rsqrt16(x):
    i = plsc.bitcast(x, jnp.int32)
    y = plsc.bitcast(jnp.int32(0x5F3759DF) - jnp.right_shift(i, 1),
                     jnp.float32)
    for _ in range(3):
        y = y * (1.5 - 0.5 * x * y * y)
    return y


def _gnn_body(src_hbm, dst_hbm, feat_hbm, fcw_hbm, params_hbm, out_hbm,
              src_v, dst_v, feat_v, fcw_v, params_v,
              x0, x1, h0, h1, onorm, inorm, acc, shared, res_v):
    w = lax.axis_index("s")
    is_lead = w == 0

    start = w * BIG - jnp.maximum(w - NBIG, 0)
    cnt = jnp.where(w < NBIG, BIG, BIG - 1)

    pltpu.sync_copy(src_hbm.at[pl.ds(start * 16, (BIG - 1) * 16)],
                    src_v.at[pl.ds(0, (BIG - 1) * 16)])
    pltpu.sync_copy(dst_hbm.at[pl.ds(start * 16, (BIG - 1) * 16)],
                    dst_v.at[pl.ds(0, (BIG - 1) * 16)])

    @pl.when(w < NBIG)
    def _():
        off = (start + BIG - 1) * 16
        pltpu.sync_copy(src_hbm.at[pl.ds(off, 16)],
                        src_v.at[pl.ds((BIG - 1) * 16, 16)])
        pltpu.sync_copy(dst_hbm.at[pl.ds(off, 16)],
                        dst_v.at[pl.ds((BIG - 1) * 16, 16)])

    pltpu.sync_copy(feat_hbm, feat_v)
    pltpu.sync_copy(params_hbm, params_v)

    @pl.when(is_lead)
    def _():
        pltpu.sync_copy(fcw_hbm, fcw_v)

    zeros16 = jnp.zeros((16,), jnp.float32)
    ones16 = jnp.ones((16,), jnp.float32)
    iota16 = lax.iota(jnp.int32, 16)

    def bc(j):
        return plsc.load_gather(params_v, [jnp.full((16,), j, jnp.int32)])

    def zero_acc(r, c):
        acc[r, pl.ds(0, 16)] = zeros16
        acc[r, pl.ds(16, 16)] = zeros16
        acc[r, pl.ds(32, 16)] = zeros16
        return c

    def reduce_acc():
        plsc.subcore_barrier()

        @pl.when(w == 0)
        def _():
            pltpu.sync_copy(acc, shared)
        plsc.subcore_barrier()

        @pl.when(w != 0)
        def _():
            pltpu.sync_copy(acc, shared.at[iota16], add=True)
        plsc.subcore_barrier()
        pltpu.sync_copy(shared, acc)

    lax.fori_loop(0, NSUB, zero_acc, 0)

    def deg(i, c):
        sl = pl.ds(i * 16, 16)
        s = src_v[sl]
        d = dst_v[sl]
        plsc.addupdate_scatter(acc, [s & 15, jnp.right_shift(s, 4)], ones16)
        plsc.addupdate_scatter(
            acc, [d & 15, jnp.right_shift(d, 4) + COL_B], ones16)
        return c
    lax.fori_loop(0, cnt, deg, 0)
    reduce_acc()

    def norm(i, c):
        sl = pl.ds(i * 16, 16)
        fi = jnp.full((16,), i, jnp.int32)
        dv0 = plsc.load_gather(acc, [iota16, fi])
        dv1 = plsc.load_gather(acc, [iota16, fi + COL_B])
        onorm[sl] = _rsqrt16(jnp.maximum(dv0, 1.0))
        inorm[sl] = _rsqrt16(jnp.maximum(dv1, 1.0))
        return c
    lax.fori_loop(0, NCH_N, norm, 0)

    def layer(get0, get1, pbase, wr0, wr1):
        w00, w01, w10, w11 = bc(pbase), bc(pbase + 1), bc(pbase + 2), bc(pbase + 3)
        b0, b1 = bc(pbase + 4), bc(pbase + 5)

        def pre(i, c):
            sl = pl.ds(i * 16, 16)
            on = onorm[sl]
            v0 = get0(i) * on
            v1 = get1(i) * on
            h0[sl] = v0 * w00 + v1 * w10
            h1[sl] = v0 * w01 + v1 * w11
            return c
        lax.fori_loop(0, NCH_N, pre, 0)
        lax.fori_loop(0, NSUB, zero_acc, 0)

        def edge(i, c):
            sl = pl.ds(i * 16, 16)
            s = src_v[sl]
            d = dst_v[sl]
            m0 = plsc.load_gather(h0, [s])
            m1 = plsc.load_gather(h1, [s])
            dlo = d & 15
            dhi = jnp.right_shift(d, 4)
            plsc.addupdate_scatter(acc, [dlo, dhi], m0)
            plsc.addupdate_scatter(acc, [dlo, dhi + COL_B], m1)
            return c
        lax.fori_loop(0, cnt, edge, 0)
        reduce_acc()

        def post(i, c):
            sl = pl.ds(i * 16, 16)
            inn = inorm[sl]
            fi = jnp.full((16,), i, jnp.int32)
            av0 = plsc.load_gather(acc, [iota16, fi])
            av1 = plsc.load_gather(acc, [iota16, fi + COL_B])
            wr0[sl] = jnp.maximum(av0 * inn + b0, 0.0)
            wr1[sl] = jnp.maximum(av1 * inn + b1, 0.0)
            return c
        lax.fori_loop(0, NCH_N, post, 0)

    layer(lambda i: plsc.load_gather(
              feat_v, [(iota16 + jnp.full((16,), i * 16, jnp.int32)) * 2]),
          lambda i: plsc.load_gather(
              feat_v, [(iota16 + jnp.full((16,), i * 16, jnp.int32)) * 2 + 1]),
          0, x0, x1)
    layer(lambda i: x0[pl.ds(i * 16, 16)],
          lambda i: x1[pl.ds(i * 16, 16)],
          6, x0, x1)

    @pl.when(is_lead)
    def _():
        def fc(i, a):
            sl = pl.ds(i * 16, 16)
            nvec = iota16 + jnp.full((16,), i * 16, jnp.int32)
            gidx = jnp.minimum(nvec, N - 1) * 2
            g0 = plsc.load_gather(fcw_v, [gidx])
            g1 = plsc.load_gather(fcw_v, [gidx + 1])
            contrib = x0[sl] * g0 + x1[sl] * g1
            return a + jnp.where(nvec < N, contrib, 0.0)
        a = lax.fori_loop(0, NCH_N, fc, zeros16)
        tot = jnp.full((16,), jnp.sum(a)) + bc(12)
        res_v[...] = 1.0 / (1.0 + jnp.exp(-tot))
        pltpu.sync_copy(res_v, out_hbm)


_gnn = functools.partial(
    pl.kernel,
    out_type=jax.ShapeDtypeStruct((16,), jnp.float32),
    mesh=plsc.VectorSubcoreMesh(core_axis_name="c", subcore_axis_name="s",
                                num_cores=1, num_subcores=16),
    compiler_params=pltpu.CompilerParams(needs_layout_passes=False),
    scratch_types=[
        pltpu.VMEM((EPT,), jnp.int32),
        pltpu.VMEM((EPT,), jnp.int32),
        pltpu.VMEM((2 * N,), jnp.float32),
        pltpu.VMEM((2 * N,), jnp.float32),
        pltpu.VMEM((16,), jnp.float32),
        pltpu.VMEM((NPAD,), jnp.float32),
        pltpu.VMEM((NPAD,), jnp.float32),
        pltpu.VMEM((NPAD,), jnp.float32),
        pltpu.VMEM((NPAD,), jnp.float32),
        pltpu.VMEM((NPAD,), jnp.float32),
        pltpu.VMEM((NPAD,), jnp.float32),
        pltpu.VMEM((NSUB, ACC_COLS), jnp.float32),
        pltpu.VMEM_SHARED((NSUB, ACC_COLS), jnp.float32),
        pltpu.VMEM((16,), jnp.float32),
    ],
)(_gnn_body)


def kernel(features, edge_index, W1, b1, W2, b2, fc_w, fc_b):
    params = jnp.concatenate([
        W1.ravel(), b1, W2.ravel(), b2, fc_b, jnp.zeros((3,), jnp.float32),
    ])
    res = _gnn(edge_index[0], edge_index[1], features.reshape(2 * N),
               fc_w.reshape(2 * N), params)
    return res[0:1].reshape(1, 1)


# speedup vs baseline: 1.0893x; 1.0893x over previous
import functools

import jax
import jax.numpy as jnp
from jax import lax
from jax.experimental import pallas as pl
from jax.experimental.pallas import tpu as pltpu
from jax.experimental.pallas import tpu_sc as plsc

N = 286
NPAD = 288
NCH_N = NPAD // 16
E = 9152
NCH_E = E // 16
NSUB = 16
BIG = 36
NBIG = NCH_E - NSUB * (BIG - 1)
EPT = BIG * 16

ACC_COLS = 36
COL_B = 18


def _rsqrt16(x):
    i = plsc.bitcast(x, jnp.int32)
    y = plsc.bitcast(jnp.int32(0x5F3759DF) - jnp.right_shift(i, 1),
                     jnp.float32)
    for _ in range(3):
        y = y * (1.5 - 0.5 * x * y * y)
    return y


def _gnn_body(src_hbm, dst_hbm, feat_hbm, fcw_hbm, params_hbm, out_hbm,
              src_v, dst_v, feat_v, fcw_v, params_v,
              x0, x1, h0, h1, onorm, inorm, acc, shared, res_v, sem):
    w = lax.axis_index("s")
    is_lead = w == 0

    start = w * BIG - jnp.maximum(w - NBIG, 0)
    cnt = jnp.where(w < NBIG, BIG, BIG - 1)

    copies = [
        pltpu.async_copy(src_hbm.at[pl.ds(start * 16, (BIG - 1) * 16)],
                         src_v.at[pl.ds(0, (BIG - 1) * 16)], sem),
        pltpu.async_copy(dst_hbm.at[pl.ds(start * 16, (BIG - 1) * 16)],
                         dst_v.at[pl.ds(0, (BIG - 1) * 16)], sem),
        pltpu.async_copy(feat_hbm, feat_v, sem),
        pltpu.async_copy(params_hbm, params_v, sem),
    ]

    @pl.when(w < NBIG)
    def _():
        off = (start + BIG - 1) * 16
        pltpu.async_copy(src_hbm.at[pl.ds(off, 16)],
                         src_v.at[pl.ds((BIG - 1) * 16, 16)], sem).wait()
        pltpu.async_copy(dst_hbm.at[pl.ds(off, 16)],
                         dst_v.at[pl.ds((BIG - 1) * 16, 16)], sem).wait()

    @pl.when(is_lead)
    def _():
        pltpu.async_copy(fcw_hbm, fcw_v, sem).wait()

    for c in copies:
        c.wait()

    zeros16 = jnp.zeros((16,), jnp.float32)
    ones16 = jnp.ones((16,), jnp.float32)
    iota16 = lax.iota(jnp.int32, 16)

    def bc(j):
        return plsc.load_gather(params_v, [jnp.full((16,), j, jnp.int32)])

    def zero_acc(r, c):
        acc[r, pl.ds(0, 16)] = zeros16
        acc[r, pl.ds(16, 16)] = zeros16
        acc[r, pl.ds(20, 16)] = zeros16
        return c

    def reduce_acc():
        plsc.subcore_barrier()

        @pl.when(w == 0)
        def _():
            pltpu.sync_copy(acc, shared)
        plsc.subcore_barrier()

        @pl.when(w != 0)
        def _():
            pltpu.sync_copy(acc, shared.at[iota16], add=True)
        plsc.subcore_barrier()
        pltpu.sync_copy(shared, acc)

    lax.fori_loop(0, NSUB, zero_acc, 0)

    def deg(i, c):
        sl = pl.ds(i * 16, 16)
        s = src_v[sl]
        d = dst_v[sl]
        plsc.addupdate_scatter(acc, [s & 15, jnp.right_shift(s, 4)], ones16)
        plsc.addupdate_scatter(
            acc, [d & 15, jnp.right_shift(d, 4) + COL_B], ones16)
        return c
    lax.fori_loop(0, cnt, deg, 0)
    reduce_acc()

    def norm(i, c):
        sl = pl.ds(i * 16, 16)
        fi = jnp.full((16,), i, jnp.int32)
        dv0 = plsc.load_gather(acc, [iota16, fi])
        dv1 = plsc.load_gather(acc, [iota16, fi + COL_B])
        onorm[sl] = _rsqrt16(jnp.maximum(dv0, 1.0))
        inorm[sl] = _rsqrt16(jnp.maximum(dv1, 1.0))
        return c
    lax.fori_loop(0, NCH_N, norm, 0)

    def layer(get0, get1, pbase, wr0, wr1):
        w00, w01, w10, w11 = bc(pbase), bc(pbase + 1), bc(pbase + 2), bc(pbase + 3)
        b0, b1 = bc(pbase + 4), bc(pbase + 5)

        def pre(i, c):
            sl = pl.ds(i * 16, 16)
            on = onorm[sl]
            v0 = get0(i) * on
            v1 = get1(i) * on
            h0[sl] = v0 * w00 + v1 * w10
            h1[sl] = v0 * w01 + v1 * w11
            return c
        lax.fori_loop(0, NCH_N, pre, 0)
        lax.fori_loop(0, NSUB, zero_acc, 0)

        def edge(i, c):
            sl = pl.ds(i * 16, 16)
            s = src_v[sl]
            d = dst_v[sl]
            m0 = plsc.load_gather(h0, [s])
            m1 = plsc.load_gather(h1, [s])
            dlo = d & 15
            dhi = jnp.right_shift(d, 4)
            plsc.addupdate_scatter(acc, [dlo, dhi], m0)
            plsc.addupdate_scatter(acc, [dlo, dhi + COL_B], m1)
            return c
        lax.fori_loop(0, cnt, edge, 0)
        reduce_acc()

        def post(i, c):
            sl = pl.ds(i * 16, 16)
            inn = inorm[sl]
            fi = jnp.full((16,), i, jnp.int32)
            av0 = plsc.load_gather(acc, [iota16, fi])
            av1 = plsc.load_gather(acc, [iota16, fi + COL_B])
            wr0[sl] = jnp.maximum(av0 * inn + b0, 0.0)
            wr1[sl] = jnp.maximum(av1 * inn + b1, 0.0)
            return c
        lax.fori_loop(0, NCH_N, post, 0)

    layer(lambda i: plsc.load_gather(
              feat_v, [(iota16 + jnp.full((16,), i * 16, jnp.int32)) * 2]),
          lambda i: plsc.load_gather(
              feat_v, [(iota16 + jnp.full((16,), i * 16, jnp.int32)) * 2 + 1]),
          0, x0, x1)
    layer(lambda i: x0[pl.ds(i * 16, 16)],
          lambda i: x1[pl.ds(i * 16, 16)],
          6, x0, x1)

    @pl.when(is_lead)
    def _():
        def fc(i, a):
            sl = pl.ds(i * 16, 16)
            nvec = iota16 + jnp.full((16,), i * 16, jnp.int32)
            gidx = jnp.minimum(nvec, N - 1) * 2
            g0 = plsc.load_gather(fcw_v, [gidx])
            g1 = plsc.load_gather(fcw_v, [gidx + 1])
            contrib = x0[sl] * g0 + x1[sl] * g1
            return a + jnp.where(nvec < N, contrib, 0.0)
        a = lax.fori_loop(0, NCH_N, fc, zeros16)
        tot = jnp.full((16,), jnp.sum(a)) + bc(12)
        res_v[...] = 1.0 / (1.0 + jnp.exp(-tot))
        pltpu.sync_copy(res_v, out_hbm)


_gnn = functools.partial(
    pl.kernel,
    out_type=jax.ShapeDtypeStruct((16,), jnp.float32),
    mesh=plsc.VectorSubcoreMesh(core_axis_name="c", subcore_axis_name="s",
                                num_cores=1, num_subcores=16),
    compiler_params=pltpu.CompilerParams(needs_layout_passes=False,
                                         disable_bounds_checks=True),
    scratch_types=[
        pltpu.VMEM((EPT,), jnp.int32),
        pltpu.VMEM((EPT,), jnp.int32),
        pltpu.VMEM((2 * N,), jnp.float32),
        pltpu.VMEM((2 * N,), jnp.float32),
        pltpu.VMEM((16,), jnp.float32),
        pltpu.VMEM((NPAD,), jnp.float32),
        pltpu.VMEM((NPAD,), jnp.float32),
        pltpu.VMEM((NPAD,), jnp.float32),
        pltpu.VMEM((NPAD,), jnp.float32),
        pltpu.VMEM((NPAD,), jnp.float32),
        pltpu.VMEM((NPAD,), jnp.float32),
        pltpu.VMEM((NSUB, ACC_COLS), jnp.float32),
        pltpu.VMEM_SHARED((NSUB, ACC_COLS), jnp.float32),
        pltpu.VMEM((16,), jnp.float32),
        pltpu.SemaphoreType.DMA,
    ],
)(_gnn_body)


def kernel(features, edge_index, W1, b1, W2, b2, fc_w, fc_b):
    params = jnp.concatenate([
        W1.ravel(), b1, W2.ravel(), b2, fc_b, jnp.zeros((3,), jnp.float32),
    ])
    res = _gnn(edge_index[0], edge_index[1], features.reshape(2 * N),
               fc_w.reshape(2 * N), params)
    return res[0:1].reshape(1, 1)


# speedup vs baseline: 1.0904x; 1.0010x over previous
import functools

import jax
import jax.numpy as jnp
from jax import lax
from jax.experimental import pallas as pl
from jax.experimental.pallas import tpu as pltpu
from jax.experimental.pallas import tpu_sc as plsc

N = 286
NPAD = 288
NCH_N = NPAD // 16
E = 9152
NCH_E = E // 16
NSUB = 16
BIG = 36
NBIG = NCH_E - NSUB * (BIG - 1)
EPT = BIG * 16

ACC_COLS = 40
COL_B = 20


def _rsqrt16(x):
    i = plsc.bitcast(x, jnp.int32)
    y = plsc.bitcast(jnp.int32(0x5F3759DF) - jnp.right_shift(i, 1),
                     jnp.float32)
    for _ in range(3):
        y = y * (1.5 - 0.5 * x * y * y)
    return y


def _gnn_body(src_hbm, dst_hbm, feat_hbm, fcw_hbm, params_hbm, out_hbm,
              src_v, dst_v, feat_v, fcw_v, params_v,
              x0, x1, h0, h1, onorm, inorm, acc, shared, res_v, sem):
    w = lax.axis_index("s")
    is_lead = w == 0

    start = w * BIG - jnp.maximum(w - NBIG, 0)
    cnt = jnp.where(w < NBIG, BIG, BIG - 1)

    copies = [
        pltpu.async_copy(src_hbm.at[pl.ds(start * 16, (BIG - 1) * 16)],
                         src_v.at[pl.ds(0, (BIG - 1) * 16)], sem),
        pltpu.async_copy(dst_hbm.at[pl.ds(start * 16, (BIG - 1) * 16)],
                         dst_v.at[pl.ds(0, (BIG - 1) * 16)], sem),
        pltpu.async_copy(feat_hbm, feat_v, sem),
        pltpu.async_copy(params_hbm, params_v, sem),
    ]

    @pl.when(w < NBIG)
    def _():
        off = (start + BIG - 1) * 16
        pltpu.async_copy(src_hbm.at[pl.ds(off, 16)],
                         src_v.at[pl.ds((BIG - 1) * 16, 16)], sem).wait()
        pltpu.async_copy(dst_hbm.at[pl.ds(off, 16)],
                         dst_v.at[pl.ds((BIG - 1) * 16, 16)], sem).wait()

    @pl.when(is_lead)
    def _():
        pltpu.async_copy(fcw_hbm, fcw_v, sem).wait()

    for c in copies:
        c.wait()

    zeros16 = jnp.zeros((16,), jnp.float32)
    ones16 = jnp.ones((16,), jnp.float32)
    iota16 = lax.iota(jnp.int32, 16)

    def bc(j):
        return plsc.load_gather(params_v, [jnp.full((16,), j, jnp.int32)])

    def zero_acc(r, c):
        acc[r, pl.ds(0, 16)] = zeros16
        acc[r, pl.ds(16, 16)] = zeros16
        acc[r, pl.ds(24, 16)] = zeros16
        return c

    def reduce_acc():
        plsc.subcore_barrier()

        @pl.when(w == 0)
        def _():
            pltpu.sync_copy(acc, shared)
        plsc.subcore_barrier()

        @pl.when(w != 0)
        def _():
            pltpu.sync_copy(acc, shared.at[iota16], add=True)
        plsc.subcore_barrier()
        pltpu.sync_copy(shared, acc)

    lax.fori_loop(0, NSUB, zero_acc, 0)

    def deg(i, c):
        sl = pl.ds(i * 16, 16)
        s = src_v[sl]
        d = dst_v[sl]
        plsc.addupdate_scatter(acc, [s & 15, jnp.right_shift(s, 4)], ones16)
        plsc.addupdate_scatter(
            acc, [d & 15, jnp.right_shift(d, 4) + COL_B], ones16)
        return c
    lax.fori_loop(0, cnt, deg, 0)
    reduce_acc()

    def norm(i, c):
        sl = pl.ds(i * 16, 16)
        fi = jnp.full((16,), i, jnp.int32)
        dv0 = plsc.load_gather(acc, [iota16, fi])
        dv1 = plsc.load_gather(acc, [iota16, fi + COL_B])
        onorm[sl] = _rsqrt16(jnp.maximum(dv0, 1.0))
        inorm[sl] = _rsqrt16(jnp.maximum(dv1, 1.0))
        return c
    lax.fori_loop(0, NCH_N, norm, 0)

    def layer(get0, get1, pbase, wr0, wr1):
        w00, w01, w10, w11 = bc(pbase), bc(pbase + 1), bc(pbase + 2), bc(pbase + 3)
        b0, b1 = bc(pbase + 4), bc(pbase + 5)

        def pre(i, c):
            sl = pl.ds(i * 16, 16)
            on = onorm[sl]
            v0 = get0(i) * on
            v1 = get1(i) * on
            h0[sl] = v0 * w00 + v1 * w10
            h1[sl] = v0 * w01 + v1 * w11
            return c
        lax.fori_loop(0, NCH_N, pre, 0)
        lax.fori_loop(0, NSUB, zero_acc, 0)

        def edge(i, c):
            sl = pl.ds(i * 16, 16)
            s = src_v[sl]
            d = dst_v[sl]
            m0 = plsc.load_gather(h0, [s])
            m1 = plsc.load_gather(h1, [s])
            dlo = d & 15
            dhi = jnp.right_shift(d, 4)
            plsc.addupdate_scatter(acc, [dlo, dhi], m0)
            plsc.addupdate_scatter(acc, [dlo, dhi + COL_B], m1)
            return c
        lax.fori_loop(0, cnt, edge, 0)
        reduce_acc()

        def post(i, c):
            sl = pl.ds(i * 16, 16)
            inn = inorm[sl]
            fi = jnp.full((16,), i, jnp.int32)
            av0 = plsc.load_gather(acc, [iota16, fi])
            av1 = plsc.load_gather(acc, [iota16, fi + COL_B])
            wr0[sl] = jnp.maximum(av0 * inn + b0, 0.0)
            wr1[sl] = jnp.maximum(av1 * inn + b1, 0.0)
            return c
        lax.fori_loop(0, NCH_N, post, 0)

    layer(lambda i: plsc.load_gather(
              feat_v, [(iota16 + jnp.full((16,), i * 16, jnp.int32)) * 2]),
          lambda i: plsc.load_gather(
              feat_v, [(iota16 + jnp.full((16,), i * 16, jnp.int32)) * 2 + 1]),
          0, x0, x1)
    layer(lambda i: x0[pl.ds(i * 16, 16)],
          lambda i: x1[pl.ds(i * 16, 16)],
          6, x0, x1)

    @pl.when(is_lead)
    def _():
        def fc(i, a):
            sl = pl.ds(i * 16, 16)
            nvec = iota16 + jnp.full((16,), i * 16, jnp.int32)
            gidx = jnp.minimum(nvec, N - 1) * 2
            g0 = plsc.load_gather(fcw_v, [gidx])
            g1 = plsc.load_gather(fcw_v, [gidx + 1])
            contrib = x0[sl] * g0 + x1[sl] * g1
            return a + jnp.where(nvec < N, contrib, 0.0)
        a = lax.fori_loop(0, NCH_N, fc, zeros16)
        tot = jnp.full((16,), jnp.sum(a)) + bc(12)
        res_v[...] = 1.0 / (1.0 + jnp.exp(-tot))
        pltpu.sync_copy(res_v, out_hbm)


_gnn = functools.partial(
    pl.kernel,
    out_type=jax.ShapeDtypeStruct((16,), jnp.float32),
    mesh=plsc.VectorSubcoreMesh(core_axis_name="c", subcore_axis_name="s",
                                num_cores=1, num_subcores=16),
    compiler_params=pltpu.CompilerParams(needs_layout_passes=False,
                                         disable_bounds_checks=True),
    scratch_types=[
        pltpu.VMEM((EPT,), jnp.int32),
        pltpu.VMEM((EPT,), jnp.int32),
        pltpu.VMEM((2 * N,), jnp.float32),
        pltpu.VMEM((2 * N,), jnp.float32),
        pltpu.VMEM((16,), jnp.float32),
        pltpu.VMEM((NPAD,), jnp.float32),
        pltpu.VMEM((NPAD,), jnp.float32),
        pltpu.VMEM((NPAD,), jnp.float32),
        pltpu.VMEM((NPAD,), jnp.float32),
        pltpu.VMEM((NPAD,), jnp.float32),
        pltpu.VMEM((NPAD,), jnp.float32),
        pltpu.VMEM((NSUB, ACC_COLS), jnp.float32),
        pltpu.VMEM_SHARED((NSUB, ACC_COLS), jnp.float32),
        pltpu.VMEM((16,), jnp.float32),
        pltpu.SemaphoreType.DMA,
    ],
)(_gnn_body)


def kernel(features, edge_index, W1, b1, W2, b2, fc_w, fc_b):
    params = jnp.concatenate([
        W1.ravel(), b1, W2.ravel(), b2, fc_b, jnp.zeros((3,), jnp.float32),
    ])
    res = _gnn(edge_index[0], edge_index[1], features.reshape(2 * N),
               fc_w.reshape(2 * N), params)
    return res[0:1].reshape(1, 1)
